# baseline (speedup 1.0000x reference)
import numpy as np

import jax
import jax.numpy as jnp
from jax.experimental import pallas as pl
from jax.experimental.pallas import tpu as pltpu

_TILE = 2048
_W = 128

_SEL_NP = np.zeros((_W, _W), np.float32)
_SEL_NP[0:10, 0:10] = 1.0
_SEL_NP[20:30, 0:10] = 1.0
_SEL_NP[10:20, 10:20] = 1.0
_SEL_NP[30:40, 10:20] = 1.0


def _router_body(x_ref, wl_ref, wr_ref, wp_ref, bl_ref, br_ref, bp_ref,
                 s_ref, o_ref):
    wl = wl_ref[...]
    wr = wr_ref[...]
    wp = wp_ref[...]
    d = wl.shape[0]
    wz = jnp.zeros((d, _W - 40), jnp.float32)
    w_cat = jnp.concatenate(
        [wl, wr, wl - wp, wr + wp, wz], axis=1).astype(jnp.bfloat16)
    bl = bl_ref[...]
    br = br_ref[...]
    bp = bp_ref[...]
    bz = jnp.zeros((1, _W - 40), jnp.float32)
    b_cat = jnp.concatenate([bl, br, bl - bp, br + bp, bz], axis=1)

    x = x_ref[...].astype(jnp.bfloat16)
    logits = jax.lax.dot_general(
        x, w_cat, (((1,), (0,)), ((), ())),
        preferred_element_type=jnp.float32,
    )
    e = jnp.exp(logits + b_cat)
    den = jax.lax.dot_general(
        e, s_ref[...], (((1,), (0,)), ((), ())),
        preferred_element_type=jnp.float32,
    )
    o_ref[...] = e[:, 0:10] / den[:, 0:10] + e[:, 10:20] / den[:, 10:20]


def kernel(x, w_router, b_router, w_left, b_left, w_right, b_right):
    n, d = x.shape
    c = w_left.shape[1]
    sel = jnp.asarray(_SEL_NP)
    grid = (n // _TILE,)
    const = lambda i: (0, 0)
    return pl.pallas_call(
        _router_body,
        grid=grid,
        in_specs=[
            pl.BlockSpec((_TILE, d), lambda i: (i, 0)),
            pl.BlockSpec((d, c), const),
            pl.BlockSpec((d, c), const),
            pl.BlockSpec((d, 1), const),
            pl.BlockSpec((1, c), const),
            pl.BlockSpec((1, c), const),
            pl.BlockSpec((1, 1), const),
            pl.BlockSpec((_W, _W), const),
        ],
        out_specs=pl.BlockSpec((_TILE, c), lambda i: (i, 0)),
        out_shape=jax.ShapeDtypeStruct((n, c), jnp.float32),
        compiler_params=pltpu.CompilerParams(
            dimension_semantics=("arbitrary",),
        ),
    )(x, w_left, w_right, w_router,
      b_left[None, :], b_right[None, :], b_router[None, :], sel)


# speedup vs baseline: 1.0232x; 1.0232x over previous
import numpy as np

import jax
import jax.numpy as jnp
from jax.experimental import pallas as pl
from jax.experimental.pallas import tpu as pltpu

_TILE = 2048
_W = 40

_SEL_NP = np.zeros((_W, _W), np.float32)
_SEL_NP[0:10, 0:10] = 1.0
_SEL_NP[20:30, 0:10] = 1.0
_SEL_NP[10:20, 10:20] = 1.0
_SEL_NP[30:40, 10:20] = 1.0


def _router_body(x_ref, w_ref, b_ref, s_ref, o_ref):
    logits = jax.lax.dot_general(
        x_ref[...], w_ref[...], (((1,), (0,)), ((), ())),
        preferred_element_type=jnp.float32,
    )
    e = jnp.exp(logits + b_ref[...])
    den = jax.lax.dot_general(
        e, s_ref[...], (((1,), (0,)), ((), ())),
        preferred_element_type=jnp.float32,
    )
    o_ref[...] = e[:, 0:10] / den[:, 0:10] + e[:, 10:20] / den[:, 10:20]


def kernel(x, w_router, b_router, w_left, b_left, w_right, b_right):
    n, d = x.shape
    c = w_left.shape[1]
    w_cat = jnp.concatenate(
        [w_left, w_right, w_left - w_router, w_right + w_router], axis=1)
    b_cat = jnp.concatenate(
        [b_left, b_right, b_left - b_router, b_right + b_router])[None, :]
    sel = jnp.asarray(_SEL_NP)
    grid = (n // _TILE,)
    return pl.pallas_call(
        _router_body,
        grid=grid,
        in_specs=[
            pl.BlockSpec((_TILE, d), lambda i: (i, 0)),
            pl.BlockSpec((d, _W), lambda i: (0, 0)),
            pl.BlockSpec((1, _W), lambda i: (0, 0)),
            pl.BlockSpec((_W, _W), lambda i: (0, 0)),
        ],
        out_specs=pl.BlockSpec((_TILE, c), lambda i: (i, 0)),
        out_shape=jax.ShapeDtypeStruct((n, c), jnp.float32),
        compiler_params=pltpu.CompilerParams(
            dimension_semantics=("arbitrary",),
        ),
    )(x, w_cat, b_cat, sel)


# speedup vs baseline: 1.0610x; 1.0369x over previous
import numpy as np

import jax
import jax.numpy as jnp
from jax.experimental import pallas as pl
from jax.experimental.pallas import tpu as pltpu

_TILE = 2048
_W = 40

_SEL_NP = np.zeros((_W, _W), np.float32)
_SEL_NP[0:10, 0:10] = 1.0
_SEL_NP[20:30, 0:10] = 1.0
_SEL_NP[10:20, 10:20] = 1.0
_SEL_NP[30:40, 10:20] = 1.0


def _router_body(x_ref, wb_ref, s_ref, o_ref):
    logits = jax.lax.dot_general(
        x_ref[...], wb_ref[0:-1, :], (((1,), (0,)), ((), ())),
        preferred_element_type=jnp.float32,
    )
    e = jnp.exp(logits + wb_ref[-1:, :])
    den = jax.lax.dot_general(
        e, s_ref[...], (((1,), (0,)), ((), ())),
        preferred_element_type=jnp.float32,
    )
    o_ref[...] = e[:, 0:10] / den[:, 0:10] + e[:, 10:20] / den[:, 10:20]


def kernel(x, w_router, b_router, w_left, b_left, w_right, b_right):
    n, d = x.shape
    c = w_left.shape[1]
    wb_cat = jnp.concatenate(
        [jnp.concatenate(
            [w_left, w_right, w_left - w_router, w_right + w_router], axis=1),
         jnp.concatenate(
            [b_left, b_right, b_left - b_router, b_right + b_router])[None, :]],
        axis=0)
    sel = jnp.asarray(_SEL_NP)
    grid = (n // _TILE,)
    return pl.pallas_call(
        _router_body,
        grid=grid,
        in_specs=[
            pl.BlockSpec((_TILE, d), lambda i: (i, 0)),
            pl.BlockSpec((d + 1, _W), lambda i: (0, 0)),
            pl.BlockSpec((_W, _W), lambda i: (0, 0)),
        ],
        out_specs=pl.BlockSpec((_TILE, c), lambda i: (i, 0)),
        out_shape=jax.ShapeDtypeStruct((n, c), jnp.float32),
        compiler_params=pltpu.CompilerParams(
            dimension_semantics=("arbitrary",),
        ),
    )(x, wb_cat, sel)
